# baseline (speedup 1.0000x reference)
import functools

import jax
import jax.numpy as jnp
from jax import lax
from jax.experimental import pallas as pl
from jax.experimental.pallas import tpu as pltpu
from jax.experimental.pallas import tpu_sc as plsc

_D = 32
_B = 16384
_NC = 2
_NS = 16
_NW = _NC * _NS
_BPW = _B // _NW
_CHUNK = 128
_NCHUNK = _BPW // _CHUNK


def _gather_body(table_hbm, idx_hbm, out_hbm, idx_v, rows_v, sem):
    wid = lax.axis_index("s") * _NC + lax.axis_index("c")
    base = wid * _BPW
    pltpu.sync_copy(idx_hbm.at[pl.ds(base, _BPW)], idx_v)
    copies = [
        pltpu.async_copy(
            table_hbm.at[idx_v.at[pl.ds(j * _CHUNK, _CHUNK)]],
            rows_v.at[pl.ds(j * _CHUNK, _CHUNK)],
            sem,
        )
        for j in range(_NCHUNK)
    ]
    for c in copies:
        c.wait()
    pltpu.sync_copy(rows_v, out_hbm.at[pl.ds(base, _BPW)])


@jax.jit
def kernel(indices, table):
    mesh = plsc.VectorSubcoreMesh(
        core_axis_name="c", subcore_axis_name="s",
        num_cores=_NC, num_subcores=_NS,
    )
    run = pl.kernel(
        _gather_body,
        out_type=jax.ShapeDtypeStruct((_B, _D), jnp.float32),
        mesh=mesh,
        scratch_types=[
            pltpu.VMEM((_BPW,), jnp.int32),
            pltpu.VMEM((_BPW, _D), jnp.float32),
            pltpu.SemaphoreType.DMA,
        ],
        compiler_params=pltpu.CompilerParams(use_tc_tiling_on_sc=False),
    )
    return run(table, indices)


# speedup vs baseline: 4.6435x; 4.6435x over previous
import functools

import jax
import jax.numpy as jnp
from jax import lax
from jax.experimental import pallas as pl
from jax.experimental.pallas import tpu as pltpu
from jax.experimental.pallas import tpu_sc as plsc

_D = 32
_B = 16384
_NC = 2
_NS = 16
_NW = _NC * _NS
_BPW = _B // _NW
_NBUF = 8
_NG = _BPW // 16


def _gather_body(tt_hbm, idx_hbm, out_hbm, idx_v,
                 b0, b1, b2, b3, b4, b5, b6, b7, slab_v, isem, gsem):
    bufs = [b0, b1, b2, b3, b4, b5, b6, b7]
    wid = lax.axis_index("s") * _NC + lax.axis_index("c")
    base = wid * _BPW
    pltpu.async_copy(idx_hbm.at[pl.ds(base, _BPW)], idx_v, isem).wait()

    row_lo = lax.iota(jnp.int32, 16)
    row_hi = row_lo + 16

    def fetch(buf, v):
        lb = pl.multiple_of((v >> 7) << 7, 128)
        pltpu.async_copy(tt_hbm.at[:, pl.ds(lb, 128)], buf, gsem)

    def drain(buf):
        pltpu.make_async_copy(tt_hbm.at[:, pl.ds(0, 128)], buf, gsem).wait()

    def extract(buf, v, col):
        l = lax.broadcast(v & 127, (16,))
        lo = plsc.load_gather(buf, [row_lo, l])
        hi = plsc.load_gather(buf, [row_hi, l])
        c16 = lax.broadcast(col, (16,))
        plsc.store_scatter(slab_v, [row_lo, c16], lo)
        plsc.store_scatter(slab_v, [row_hi, c16], hi)

    vs0 = idx_v[pl.ds(0, 16)]
    for j in range(_NBUF):
        fetch(bufs[j], vs0[j])

    def group(g):
        off = g * 16
        vs = idx_v[pl.ds(off, 16)]
        gn = jnp.minimum(g + 1, _NG - 1)
        vs1 = idx_v[pl.ds(gn * 16, 16)]
        for j in range(16):
            buf = bufs[j % _NBUF]
            drain(buf)
            extract(buf, vs[j], off + j)
            vnext = vs[j + _NBUF] if j < _NBUF else vs1[j - _NBUF]

            @pl.when(off + j + _NBUF < _BPW)
            def _(buf=buf, vnext=vnext):
                fetch(buf, vnext)

    pl.loop(0, _NG)(group)
    pltpu.sync_copy(slab_v, out_hbm.at[:, pl.ds(base, _BPW)])


@jax.jit
def kernel(indices, table):
    mesh = plsc.VectorSubcoreMesh(
        core_axis_name="c", subcore_axis_name="s",
        num_cores=_NC, num_subcores=_NS,
    )
    run = pl.kernel(
        _gather_body,
        out_type=jax.ShapeDtypeStruct((_D, _B), jnp.float32),
        mesh=mesh,
        scratch_types=(
            [pltpu.VMEM((_BPW,), jnp.int32)]
            + [pltpu.VMEM((_D, 128), jnp.float32) for _ in range(_NBUF)]
            + [
                pltpu.VMEM((_D, _BPW), jnp.float32),
                pltpu.SemaphoreType.DMA,
                pltpu.SemaphoreType.DMA,
            ]
        ),
        compiler_params=pltpu.CompilerParams(needs_layout_passes=False),
    )
    out_t = run(table.T, indices)
    return out_t.T


# speedup vs baseline: 4.6518x; 1.0018x over previous
import functools

import jax
import jax.numpy as jnp
from jax import lax
from jax.experimental import pallas as pl
from jax.experimental.pallas import tpu as pltpu
from jax.experimental.pallas import tpu_sc as plsc

_D = 32
_B = 16384
_NC = 2
_NS = 16
_NW = _NC * _NS
_BPW = _B // _NW
_NBUF = 16
_NG = _BPW // 16


def _gather_body(tt_hbm, idx_hbm, out_hbm, idx_v,
                 b0, b1, b2, b3, b4, b5, b6, b7,
                 b8, b9, b10, b11, b12, b13, b14, b15, slab_v, isem, gsem):
    bufs = [b0, b1, b2, b3, b4, b5, b6, b7,
            b8, b9, b10, b11, b12, b13, b14, b15]
    wid = lax.axis_index("s") * _NC + lax.axis_index("c")
    base = wid * _BPW
    pltpu.async_copy(idx_hbm.at[pl.ds(base, _BPW)], idx_v, isem).wait()

    row_lo = lax.iota(jnp.int32, 16)
    row_hi = row_lo + 16

    def fetch(buf, v):
        lb = pl.multiple_of((v >> 7) << 7, 128)
        pltpu.async_copy(tt_hbm.at[:, pl.ds(lb, 128)], buf, gsem)

    def drain(buf):
        pltpu.make_async_copy(tt_hbm.at[:, pl.ds(0, 128)], buf, gsem).wait()

    def extract(buf, v, col):
        l = lax.broadcast(v & 127, (16,))
        lo = plsc.load_gather(buf, [row_lo, l])
        hi = plsc.load_gather(buf, [row_hi, l])
        c16 = lax.broadcast(col, (16,))
        plsc.store_scatter(slab_v, [row_lo, c16], lo)
        plsc.store_scatter(slab_v, [row_hi, c16], hi)

    vs0 = idx_v[pl.ds(0, 16)]
    for j in range(16):
        fetch(bufs[j], vs0[j])

    def group(g):
        off = g * 16
        vs = idx_v[pl.ds(off, 16)]
        gn = jnp.minimum(g + 1, _NG - 1)
        vs1 = idx_v[pl.ds(gn * 16, 16)]
        for j in range(16):
            buf = bufs[j]
            drain(buf)
            extract(buf, vs[j], off + j)

            @pl.when(g + 1 < _NG)
            def _(buf=buf, vnext=vs1[j]):
                fetch(buf, vnext)

    pl.loop(0, _NG)(group)
    pltpu.sync_copy(slab_v, out_hbm.at[:, pl.ds(base, _BPW)])


@jax.jit
def kernel(indices, table):
    mesh = plsc.VectorSubcoreMesh(
        core_axis_name="c", subcore_axis_name="s",
        num_cores=_NC, num_subcores=_NS,
    )
    run = pl.kernel(
        _gather_body,
        out_type=jax.ShapeDtypeStruct((_D, _B), jnp.float32),
        mesh=mesh,
        scratch_types=(
            [pltpu.VMEM((_BPW,), jnp.int32)]
            + [pltpu.VMEM((_D, 128), jnp.float32) for _ in range(_NBUF)]
            + [
                pltpu.VMEM((_D, _BPW), jnp.float32),
                pltpu.SemaphoreType.DMA,
                pltpu.SemaphoreType.DMA,
            ]
        ),
        compiler_params=pltpu.CompilerParams(needs_layout_passes=False),
    )
    out_t = run(table.T, indices)
    return out_t.T
